# Initial kernel scaffold; baseline (speedup 1.0000x reference)
import jax
import jax.numpy as jnp
from jax.experimental import pallas as pl


def kernel(x, edge_index, edge_weight, W0, b0, W1, b1, W2, b2, Wd1, bd1, Wd2, bd2):
    raise NotImplementedError("write your pallas kernel here")


# speedup vs baseline: 6.4727x; 6.4727x over previous
import functools

import jax
import jax.numpy as jnp
from jax import lax
from jax.experimental import pallas as pl
from jax.experimental.pallas import tpu as pltpu
from jax.experimental.pallas import tpu_sc as plsc

N = 10000
D = 128
H = 128
HID = 256
NUM_LABELS = 7
E = 320000

NC = 2
NS = 16
NW = NC * NS
CK = 128
NCHUNK = 79
EPT = CK * NCHUNK
E_PAD = EPT * NW
NDEG = 10240
DEG_PT = NDEG // NS
NROW = 10240
ROWS_PT = NROW // NS

_mesh = plsc.VectorSubcoreMesh(core_axis_name="c", subcore_axis_name="s")


def _zero_vmem_2d(ref, nrows):
    z = jnp.zeros((16,), jnp.float32)

    def body(i, _):
        for j in range(8):
            ref[i, pl.ds(j * 16, 16)] = z
        return 0

    lax.fori_loop(0, nrows, body, 0)


def _zero_vmem_1d(ref, n):
    z = jnp.zeros((16,), jnp.float32)

    def body(i, _):
        ref[pl.ds(i * 16, 16)] = z
        return 0

    lax.fori_loop(0, n // 16, body, 0)


@functools.partial(
    pl.kernel,
    out_type=jax.ShapeDtypeStruct((NC, NDEG), jnp.float32),
    mesh=_mesh,
    scratch_types=[
        pltpu.VMEM_SHARED((NDEG,), jnp.float32),
        pltpu.VMEM((CK,), jnp.int32),
        pltpu.VMEM((CK,), jnp.float32),
        pltpu.VMEM((DEG_PT,), jnp.float32),
    ],
)
def _deg_kernel(r_hbm, w_hbm, out, deg_sp, ridx, wbuf, zbuf):
    cid = lax.axis_index("c")
    sid = lax.axis_index("s")
    wid = sid * NC + cid

    _zero_vmem_1d(zbuf, DEG_PT)
    pltpu.sync_copy(zbuf, deg_sp.at[pl.ds(sid * DEG_PT, DEG_PT)])
    plsc.subcore_barrier()

    def chunk(k, _):
        base = wid * EPT + k * CK
        pltpu.sync_copy(r_hbm.at[pl.ds(base, CK)], ridx)
        pltpu.sync_copy(w_hbm.at[pl.ds(base, CK)], wbuf)
        pltpu.sync_copy(wbuf, deg_sp.at[ridx], add=True)
        return 0

    lax.fori_loop(0, NCHUNK, chunk, 0)
    plsc.subcore_barrier()
    pltpu.sync_copy(
        deg_sp.at[pl.ds(sid * DEG_PT, DEG_PT)],
        out.at[cid, pl.ds(sid * DEG_PT, DEG_PT)],
    )


@functools.partial(
    pl.kernel,
    out_type=jax.ShapeDtypeStruct((NC, NROW, H), jnp.float32),
    mesh=_mesh,
    scratch_types=[
        pltpu.VMEM_SHARED((NROW, H), jnp.float32),
        pltpu.VMEM((CK,), jnp.int32),
        pltpu.VMEM((CK,), jnp.int32),
        pltpu.VMEM((CK + 16,), jnp.float32),
        pltpu.VMEM((CK, H), jnp.float32),
        pltpu.SemaphoreType.DMA,
    ],
)
def _agg_kernel(hws_hbm, r_hbm, c_hbm, w_hbm, out, acc_sp, ridx, cidx, wbuf,
                rows, sem):
    cid = lax.axis_index("c")
    sid = lax.axis_index("s")
    wid = sid * NC + cid

    _zero_vmem_2d(rows, CK)
    base_row = sid * ROWS_PT
    for j in range(ROWS_PT // CK):
        pltpu.sync_copy(rows, acc_sp.at[pl.ds(base_row + j * CK, CK), :])
    plsc.subcore_barrier()

    def chunk(k, _):
        base = wid * EPT + k * CK
        pltpu.sync_copy(c_hbm.at[pl.ds(base, CK)], cidx)
        pltpu.sync_copy(r_hbm.at[pl.ds(base, CK)], ridx)
        pltpu.sync_copy(w_hbm.at[pl.ds(base, CK)], wbuf.at[pl.ds(0, CK)])
        pltpu.async_copy(hws_hbm.at[cidx], rows, sem).wait()

        def scale(e, _):
            ws = wbuf[pl.ds(e, 16)][0]
            for j in range(8):
                sl = pl.ds(j * 16, 16)
                rows[e, sl] = rows[e, sl] * ws
            return 0

        lax.fori_loop(0, CK, scale, 0)
        pltpu.sync_copy(rows, acc_sp.at[ridx], add=True)
        return 0

    lax.fori_loop(0, NCHUNK, chunk, 0)
    plsc.subcore_barrier()

    for j in range(ROWS_PT // CK):
        pltpu.sync_copy(acc_sp.at[pl.ds(base_row + j * CK, CK), :],
                        out.at[cid, pl.ds(base_row + j * CK, CK), :])


RB = 400
GRID = N // RB


def _mm1_body(x_ref, w_ref, d0_ref, d1_ref, hws_ref, dinv_ref):
    dinv = lax.rsqrt(d0_ref[...] + d1_ref[...] + 1.0)
    hw = jnp.dot(x_ref[...], w_ref[...], preferred_element_type=jnp.float32)
    hws_ref[...] = dinv * hw
    dinv_ref[...] = dinv


def _mm1(x, W0, d0, d1):
    return pl.pallas_call(
        _mm1_body,
        grid=(GRID,),
        in_specs=[
            pl.BlockSpec((RB, D), lambda i: (i, 0)),
            pl.BlockSpec((D, H), lambda i: (0, 0)),
            pl.BlockSpec((RB, 1), lambda i: (i, 0)),
            pl.BlockSpec((RB, 1), lambda i: (i, 0)),
        ],
        out_specs=[
            pl.BlockSpec((RB, H), lambda i: (i, 0)),
            pl.BlockSpec((RB, 1), lambda i: (i, 0)),
        ],
        out_shape=[
            jax.ShapeDtypeStruct((N, H), jnp.float32),
            jax.ShapeDtypeStruct((N, 1), jnp.float32),
        ],
    )(x, W0, d0, d1)


def _combine_mm_body(p0_ref, p1_ref, hws_ref, dinv_ref, b_ref, w_ref, out_ref):
    dinv = dinv_ref[...]
    h = jax.nn.relu(dinv * (p0_ref[0] + p1_ref[0] + hws_ref[...])
                    + b_ref[...])
    out_ref[...] = dinv * jnp.dot(h, w_ref[...],
                                  preferred_element_type=jnp.float32)


def _combine_mm(p, hws, dinv, b, W):
    return pl.pallas_call(
        _combine_mm_body,
        grid=(GRID,),
        in_specs=[
            pl.BlockSpec((1, RB, H), lambda i: (0, i, 0)),
            pl.BlockSpec((1, RB, H), lambda i: (1, i, 0)),
            pl.BlockSpec((RB, H), lambda i: (i, 0)),
            pl.BlockSpec((RB, 1), lambda i: (i, 0)),
            pl.BlockSpec((1, H), lambda i: (0, 0)),
            pl.BlockSpec((H, H), lambda i: (0, 0)),
        ],
        out_specs=pl.BlockSpec((RB, H), lambda i: (i, 0)),
        out_shape=jax.ShapeDtypeStruct((N, H), jnp.float32),
    )(p, p, hws, dinv, b, W)


def _final_body(p0_ref, p1_ref, hws_ref, dinv_ref, b_ref, wd1_ref, bd1_ref,
                wd2_ref, bd2_ref, out_ref):
    dinv = dinv_ref[...]
    h = jax.nn.relu(dinv * (p0_ref[0] + p1_ref[0] + hws_ref[...])
                    + b_ref[...])
    t = jax.nn.relu(jnp.dot(h, wd1_ref[...],
                            preferred_element_type=jnp.float32) + bd1_ref[...])
    out_ref[...] = jnp.dot(t, wd2_ref[...],
                           preferred_element_type=jnp.float32) + bd2_ref[...]


def _final(p, hws, dinv, b, Wd1, bd1, Wd2p, bd2p):
    return pl.pallas_call(
        _final_body,
        grid=(GRID,),
        in_specs=[
            pl.BlockSpec((1, RB, H), lambda i: (0, i, 0)),
            pl.BlockSpec((1, RB, H), lambda i: (1, i, 0)),
            pl.BlockSpec((RB, H), lambda i: (i, 0)),
            pl.BlockSpec((RB, 1), lambda i: (i, 0)),
            pl.BlockSpec((1, H), lambda i: (0, 0)),
            pl.BlockSpec((H, HID), lambda i: (0, 0)),
            pl.BlockSpec((1, HID), lambda i: (0, 0)),
            pl.BlockSpec((HID, H), lambda i: (0, 0)),
            pl.BlockSpec((1, H), lambda i: (0, 0)),
        ],
        out_specs=pl.BlockSpec((RB, H), lambda i: (i, 0)),
        out_shape=jax.ShapeDtypeStruct((N, H), jnp.float32),
    )(p, p, hws, dinv, b, Wd1, bd1, Wd2p, bd2p)


@jax.jit
def kernel(x, edge_index, edge_weight, W0, b0, W1, b1, W2, b2, Wd1, bd1,
           Wd2, bd2):
    r = edge_index[0].astype(jnp.int32)
    c = edge_index[1].astype(jnp.int32)
    w = edge_weight.astype(jnp.float32)
    pad = E_PAD - E
    r = jnp.pad(r, (0, pad))
    c = jnp.pad(c, (0, pad))
    w = jnp.pad(w, (0, pad))

    degp = _deg_kernel(r, w)
    d0 = degp[0, :N, None]
    d1 = degp[1, :N, None]

    hws, dinv = _mm1(x, W0, d0, d1)

    p = _agg_kernel(hws, r, c, w)
    hws = _combine_mm(p, hws, dinv, b0.reshape(1, H), W1)

    p = _agg_kernel(hws, r, c, w)
    hws = _combine_mm(p, hws, dinv, b1.reshape(1, H), W2)

    p = _agg_kernel(hws, r, c, w)
    Wd2p = jnp.pad(Wd2, ((0, 0), (0, H - NUM_LABELS)))
    bd2p = jnp.pad(bd2, (0, H - NUM_LABELS)).reshape(1, H)
    out = _final(p, hws, dinv, b2.reshape(1, H), Wd1,
                 bd1.reshape(1, HID), Wd2p, bd2p)
    return out[:, :NUM_LABELS]
